# Initial kernel scaffold; baseline (speedup 1.0000x reference)
import functools

import jax
import jax.numpy as jnp
from jax import lax
from jax.experimental import pallas as pl
from jax.experimental.pallas import tpu as pltpu
from jax.experimental.pallas import tpu_sc as plsc

NC = 2
NS = 16
NW = NC * NS
CHUNK = 128


def _make_seg_sum(n_pad, d, n_chunks, count_deg):
    mesh = plsc.VectorSubcoreMesh(core_axis_name="c", subcore_axis_name="s")
    rows_per_tile = n_pad // NS

    out_type = [jax.ShapeDtypeStruct((NC, n_pad, d), jnp.float32)]
    scratch = [
        pltpu.VMEM((CHUNK,), jnp.int32),
        pltpu.VMEM((CHUNK,), jnp.int32),
        pltpu.VMEM((CHUNK, d), jnp.float32),
        pltpu.VMEM_SHARED((n_pad, d), jnp.float32),
    ]
    if count_deg:
        out_type.append(jax.ShapeDtypeStruct((NC, n_pad, 16), jnp.float32))
        scratch += [
            pltpu.VMEM((CHUNK, 16), jnp.float32),
            pltpu.VMEM_SHARED((n_pad, 16), jnp.float32),
        ]

    @functools.partial(
        pl.kernel, mesh=mesh, out_type=tuple(out_type),
        scratch_types=tuple(scratch),
    )
    def seg_sum(*refs):
        if count_deg:
            (table, srcp, dstp, z_d, z_16, ones_hbm,
             out_sum, out_deg, idx_s, idx_d, rows, acc, ones_v, dacc) = refs
        else:
            (table, srcp, dstp, z_d,
             out_sum, idx_s, idx_d, rows, acc) = refs
        c = lax.axis_index("c")
        s = lax.axis_index("s")
        w = s * NC + c
        pltpu.sync_copy(z_d.at[pl.ds(0, rows_per_tile)],
                        acc.at[pl.ds(s * rows_per_tile, rows_per_tile)])
        if count_deg:
            pltpu.sync_copy(z_16.at[pl.ds(0, rows_per_tile)],
                            dacc.at[pl.ds(s * rows_per_tile, rows_per_tile)])
            pltpu.sync_copy(ones_hbm, ones_v)
        plsc.subcore_barrier()

        def body(j, carry):
            off = (w * n_chunks + j) * CHUNK
            pltpu.sync_copy(srcp.at[pl.ds(off, CHUNK)], idx_s)
            pltpu.sync_copy(dstp.at[pl.ds(off, CHUNK)], idx_d)
            pltpu.sync_copy(table.at[idx_s], rows)
            pltpu.sync_copy(rows, acc.at[idx_d], add=True)
            if count_deg:
                pltpu.sync_copy(ones_v, dacc.at[idx_d], add=True)
            return carry

        lax.fori_loop(0, n_chunks, body, 0)
        plsc.subcore_barrier()
        sl = pl.ds(s * rows_per_tile, rows_per_tile)
        pltpu.sync_copy(acc.at[sl], out_sum.at[c, sl])
        if count_deg:
            pltpu.sync_copy(dacc.at[sl], out_deg.at[c, sl])

    return seg_sum


def _sage_block(x_ref, xsum_ref, degt_ref, w1l_ref, w1r_ref, b1_ref,
                w2l_ref, w2r_ref, p_ref, q_ref):
    xs = xsum_ref[0] + xsum_ref[1]
    deg = degt_ref[0, :, 0:1] + degt_ref[1, :, 0:1]
    agg = xs * (1.0 / jnp.maximum(deg, 1.0))
    h = (jnp.dot(agg, w1l_ref[...], preferred_element_type=jnp.float32)
         + jnp.dot(x_ref[...], w1r_ref[...], preferred_element_type=jnp.float32)
         + b1_ref[...])
    h = jnp.maximum(h, 0.0)
    p_ref[...] = jnp.dot(h, w2l_ref[...], preferred_element_type=jnp.float32)
    q_ref[...] = jnp.dot(h, w2r_ref[...], preferred_element_type=jnp.float32)


def _out_block(psum_ref, q_ref, degt_ref, b2_ref, o_ref):
    ps = psum_ref[0] + psum_ref[1]
    deg = degt_ref[0, :, 0:1] + degt_ref[1, :, 0:1]
    z = ps * (1.0 / jnp.maximum(deg, 1.0)) + q_ref[...] + b2_ref[...]
    m = jnp.max(z, axis=1, keepdims=True)
    e = z - m
    o_ref[...] = e - jnp.log(jnp.sum(jnp.exp(e), axis=1, keepdims=True))


def kernel(x, W1l, W1r, b1, W2l, W2r, b2, edge_index):
    n, d_in = x.shape
    d_h = W1l.shape[1]
    d_out = W2l.shape[1]
    e = edge_index.shape[1]
    n_pad = n + 16

    n_chunks = -(-e // (NW * CHUNK))
    e_pad = NW * n_chunks * CHUNK
    src = jnp.concatenate([edge_index[0], jnp.zeros((e_pad - e,), jnp.int32)])
    dst = jnp.concatenate([edge_index[1], jnp.full((e_pad - e,), n, jnp.int32)])

    z128 = jnp.zeros((n_pad // NS, d_in), jnp.float32)
    z64 = jnp.zeros((n_pad // NS, d_out), jnp.float32)
    z16 = jnp.zeros((n_pad // NS, 16), jnp.float32)
    ones = jnp.ones((CHUNK, 16), jnp.float32)

    seg1 = _make_seg_sum(n_pad, d_in, n_chunks, count_deg=True)
    xsum, degt = seg1(x, src, dst, z128, z16, ones)

    rb = 400
    grid = (n // rb,)
    p, q = pl.pallas_call(
        _sage_block,
        grid=grid,
        in_specs=[
            pl.BlockSpec((rb, d_in), lambda i: (i, 0)),
            pl.BlockSpec((NC, rb, d_in), lambda i: (0, i, 0)),
            pl.BlockSpec((NC, rb, 16), lambda i: (0, i, 0)),
            pl.BlockSpec((d_in, d_h), lambda i: (0, 0)),
            pl.BlockSpec((d_in, d_h), lambda i: (0, 0)),
            pl.BlockSpec((1, d_h), lambda i: (0, 0)),
            pl.BlockSpec((d_h, d_out), lambda i: (0, 0)),
            pl.BlockSpec((d_h, d_out), lambda i: (0, 0)),
        ],
        out_specs=[
            pl.BlockSpec((rb, d_out), lambda i: (i, 0)),
            pl.BlockSpec((rb, d_out), lambda i: (i, 0)),
        ],
        out_shape=[
            jax.ShapeDtypeStruct((n, d_out), jnp.float32),
            jax.ShapeDtypeStruct((n, d_out), jnp.float32),
        ],
    )(x, xsum, degt, W1l, W1r, b1.reshape(1, d_h), W2l, W2r)

    seg2 = _make_seg_sum(n_pad, d_out, n_chunks, count_deg=False)
    (psum,) = seg2(p, src, dst, z64)

    out = pl.pallas_call(
        _out_block,
        grid=grid,
        in_specs=[
            pl.BlockSpec((NC, rb, d_out), lambda i: (0, i, 0)),
            pl.BlockSpec((rb, d_out), lambda i: (i, 0)),
            pl.BlockSpec((NC, rb, 16), lambda i: (0, i, 0)),
            pl.BlockSpec((1, d_out), lambda i: (0, 0)),
        ],
        out_specs=pl.BlockSpec((rb, d_out), lambda i: (i, 0)),
        out_shape=jax.ShapeDtypeStruct((n, d_out), jnp.float32),
    )(psum, q, degt, b2.reshape(1, d_out))
    return out


# speedup vs baseline: 3.3690x; 3.3690x over previous
import functools

import jax
import jax.numpy as jnp
from jax import lax
from jax.experimental import pallas as pl
from jax.experimental.pallas import tpu as pltpu
from jax.experimental.pallas import tpu_sc as plsc

NC = 2
NS = 16
NW = NC * NS
CHUNK = 128


def _pad_edges(src, dst, n, workers, e):
    n_chunks = -(-e // (workers * CHUNK))
    e_pad = workers * n_chunks * CHUNK
    srcp = jnp.concatenate([src, jnp.zeros((e_pad - e,), jnp.int32)])
    dstp = jnp.concatenate([dst, jnp.full((e_pad - e,), n, jnp.int32)])
    return srcp, dstp, n_chunks


def _make_agg_deg(n_pad, d, n_chunks):
    mesh = plsc.VectorSubcoreMesh(core_axis_name="c", subcore_axis_name="s")
    rpt = n_pad // NS

    @functools.partial(
        pl.kernel, mesh=mesh,
        out_type=(
            jax.ShapeDtypeStruct((n_pad, d), jnp.float32),
            jax.ShapeDtypeStruct((n_pad, d), jnp.float32),
        ),
        scratch_types=(
            pltpu.VMEM((CHUNK,), jnp.int32),
            pltpu.VMEM((CHUNK,), jnp.int32),
            pltpu.VMEM((CHUNK, d), jnp.float32),
            pltpu.VMEM_SHARED((n_pad, d), jnp.float32),
            pltpu.SemaphoreType.DMA,
        ),
    )
    def agg(table, srcp, dstp, z_d, ones_hbm, out_sum, out_deg,
            idx_s, idx_d, rows, acc, gsem):
        c = lax.axis_index("c")
        s = lax.axis_index("s")
        pltpu.sync_copy(z_d.at[pl.ds(0, rpt)], acc.at[pl.ds(s * rpt, rpt)])
        plsc.subcore_barrier()
        sl = pl.ds(s * rpt, rpt)

        @pl.when(c == 0)
        def _x_role():
            def body(j, carry):
                off = (s * n_chunks + j) * CHUNK
                pltpu.sync_copy(srcp.at[pl.ds(off, CHUNK)], idx_s)
                pltpu.sync_copy(dstp.at[pl.ds(off, CHUNK)], idx_d)
                pltpu.async_copy(table.at[idx_s], rows, gsem).wait()
                pltpu.sync_copy(rows, acc.at[idx_d], add=True)
                return carry
            lax.fori_loop(0, n_chunks, body, 0)
            plsc.subcore_barrier()
            pltpu.sync_copy(acc.at[sl], out_sum.at[sl])

        @pl.when(c == 1)
        def _deg_role():
            pltpu.sync_copy(ones_hbm, rows)
            def body(j, carry):
                off = (s * n_chunks + j) * CHUNK
                pltpu.sync_copy(dstp.at[pl.ds(off, CHUNK)], idx_d)
                pltpu.sync_copy(rows, acc.at[idx_d], add=True)
                return carry
            lax.fori_loop(0, n_chunks, body, 0)
            plsc.subcore_barrier()
            pltpu.sync_copy(acc.at[sl], out_deg.at[sl])

    return agg


def _make_seg_sum(n_pad, d, n_chunks):
    mesh = plsc.VectorSubcoreMesh(core_axis_name="c", subcore_axis_name="s")
    rpt = n_pad // NS

    @functools.partial(
        pl.kernel, mesh=mesh,
        out_type=(jax.ShapeDtypeStruct((NC, n_pad, d), jnp.float32),),
        scratch_types=(
            pltpu.VMEM((CHUNK,), jnp.int32),
            pltpu.VMEM((CHUNK,), jnp.int32),
            pltpu.VMEM((CHUNK, d), jnp.float32),
            pltpu.VMEM_SHARED((n_pad, d), jnp.float32),
            pltpu.SemaphoreType.DMA,
        ),
    )
    def seg(table, srcp, dstp, z_d, out_sum,
            idx_s, idx_d, rows, acc, gsem):
        c = lax.axis_index("c")
        s = lax.axis_index("s")
        w = s * NC + c
        pltpu.sync_copy(z_d.at[pl.ds(0, rpt)], acc.at[pl.ds(s * rpt, rpt)])
        plsc.subcore_barrier()

        def body(j, carry):
            off = (w * n_chunks + j) * CHUNK
            pltpu.sync_copy(srcp.at[pl.ds(off, CHUNK)], idx_s)
            pltpu.sync_copy(dstp.at[pl.ds(off, CHUNK)], idx_d)
            pltpu.async_copy(table.at[idx_s], rows, gsem).wait()
            pltpu.sync_copy(rows, acc.at[idx_d], add=True)
            return carry

        lax.fori_loop(0, n_chunks, body, 0)
        plsc.subcore_barrier()
        sl = pl.ds(s * rpt, rpt)
        pltpu.sync_copy(acc.at[sl], out_sum.at[c, sl])

    return seg


def _sage_block(x_ref, xsum_ref, degf_ref, w1l_ref, w1r_ref, b1_ref,
                w2l_ref, w2r_ref, pq_ref):
    rdeg = 1.0 / jnp.maximum(degf_ref[...], 1.0)
    agg = xsum_ref[...] * rdeg
    h = (jnp.dot(agg, w1l_ref[...], preferred_element_type=jnp.float32)
         + jnp.dot(x_ref[...], w1r_ref[...], preferred_element_type=jnp.float32)
         + b1_ref[...])
    h = jnp.maximum(h, 0.0)
    p = jnp.dot(h, w2l_ref[...], preferred_element_type=jnp.float32)
    q = jnp.dot(h, w2r_ref[...], preferred_element_type=jnp.float32)
    pq_ref[...] = jnp.concatenate([p, q], axis=1)


def _out_block(psum_ref, pq_ref, degf_ref, b2_ref, o_ref):
    d_out = o_ref.shape[1]
    ps = (psum_ref[0] + psum_ref[1]) * (1.0 / jnp.maximum(degf_ref[...], 1.0))
    z = ps[:, :d_out] + pq_ref[...][:, d_out:] + b2_ref[...]
    m = jnp.max(z, axis=1, keepdims=True)
    e = z - m
    o_ref[...] = e - jnp.log(jnp.sum(jnp.exp(e), axis=1, keepdims=True))


def kernel(x, W1l, W1r, b1, W2l, W2r, b2, edge_index):
    n, d_in = x.shape
    d_h = W1l.shape[1]
    d_out = W2l.shape[1]
    e = edge_index.shape[1]
    n_pad = -(-(n + 1) // 128) * 128

    src16, dst16, nch16 = _pad_edges(edge_index[0], edge_index[1], n, NS, e)
    src32, dst32, nch32 = _pad_edges(edge_index[0], edge_index[1], n, NW, e)

    z128 = jnp.zeros((n_pad // NS, d_in), jnp.float32)
    ones = jnp.ones((CHUNK, d_in), jnp.float32)

    agg1 = _make_agg_deg(n_pad, d_in, nch16)
    xsum, degf = agg1(x, src16, dst16, z128, ones)

    rb = 400
    grid = (n // rb,)
    pq = pl.pallas_call(
        _sage_block,
        grid=grid,
        in_specs=[
            pl.BlockSpec((rb, d_in), lambda i: (i, 0)),
            pl.BlockSpec((rb, d_in), lambda i: (i, 0)),
            pl.BlockSpec((rb, d_in), lambda i: (i, 0)),
            pl.BlockSpec((d_in, d_h), lambda i: (0, 0)),
            pl.BlockSpec((d_in, d_h), lambda i: (0, 0)),
            pl.BlockSpec((1, d_h), lambda i: (0, 0)),
            pl.BlockSpec((d_h, d_out), lambda i: (0, 0)),
            pl.BlockSpec((d_h, d_out), lambda i: (0, 0)),
        ],
        out_specs=pl.BlockSpec((rb, 2 * d_out), lambda i: (i, 0)),
        out_shape=jax.ShapeDtypeStruct((n, 2 * d_out), jnp.float32),
    )(x, xsum, degf, W1l, W1r, b1.reshape(1, d_h), W2l, W2r)

    seg2 = _make_seg_sum(n_pad, 2 * d_out, nch32)
    (psum,) = seg2(pq, src32, dst32, z128)

    out = pl.pallas_call(
        _out_block,
        grid=grid,
        in_specs=[
            pl.BlockSpec((NC, rb, 2 * d_out), lambda i: (0, i, 0)),
            pl.BlockSpec((rb, 2 * d_out), lambda i: (i, 0)),
            pl.BlockSpec((rb, d_in), lambda i: (i, 0)),
            pl.BlockSpec((1, d_out), lambda i: (0, 0)),
        ],
        out_specs=pl.BlockSpec((rb, d_out), lambda i: (i, 0)),
        out_shape=jax.ShapeDtypeStruct((n, d_out), jnp.float32),
    )(psum, pq, degf, b2.reshape(1, d_out))
    return out
